# baseline (speedup 1.0000x reference)
import functools

import jax
import jax.numpy as jnp
from jax import lax
from jax.experimental import pallas as pl
from jax.experimental.pallas import tpu as pltpu
from jax.experimental.pallas import tpu_sc as plsc

NUM_CORES = 2
NUM_SUBCORES = 16
NUM_WORKERS = NUM_CORES * NUM_SUBCORES
CHUNK = 128


def _emb_lookup(n_total: int, emb_dim: int):
    n_per_w = n_total // NUM_WORKERS
    n_chunks = n_per_w // CHUNK
    mesh = plsc.VectorSubcoreMesh(core_axis_name="c", subcore_axis_name="s")

    @functools.partial(
        pl.kernel,
        mesh=mesh,
        out_type=jax.ShapeDtypeStruct((n_total, emb_dim), jnp.float32),
        scratch_types=[
            pltpu.VMEM((CHUNK,), jnp.int32),
            pltpu.VMEM((CHUNK, emb_dim), jnp.float32),
            pltpu.SemaphoreType.DMA,
        ],
        compiler_params=pltpu.CompilerParams(use_tc_tiling_on_sc=False),
    )
    def body(idx_hbm, table_hbm, out_hbm, idx_v, rows_v, sem):
        wid = lax.axis_index("s") * NUM_CORES + lax.axis_index("c")
        base = wid * n_per_w

        def step(c, carry):
            off = base + c * CHUNK
            pltpu.sync_copy(idx_hbm.at[pl.ds(off, CHUNK)], idx_v)
            pltpu.async_copy(table_hbm.at[idx_v], rows_v, sem).wait()
            pltpu.sync_copy(rows_v, out_hbm.at[pl.ds(off, CHUNK)])
            return carry

        lax.fori_loop(0, n_chunks, step, 0)

    return body


def kernel(indices, table):
    batch, hist = indices.shape
    _, emb_dim = table.shape
    n_total = batch * hist
    idx_flat = indices.reshape(n_total).astype(jnp.int32)
    out = _emb_lookup(n_total, emb_dim)(idx_flat, table)
    return out.reshape(batch, hist, emb_dim)


# speedup vs baseline: 1.0710x; 1.0710x over previous
import functools

import jax
import jax.numpy as jnp
from jax import lax
from jax.experimental import pallas as pl
from jax.experimental.pallas import tpu as pltpu
from jax.experimental.pallas import tpu_sc as plsc

NUM_CORES = 2
NUM_SUBCORES = 16
NUM_WORKERS = NUM_CORES * NUM_SUBCORES
CHUNK = 128
NBUF = 10


def _emb_lookup(n_total: int, emb_dim: int):
    n_per_w = n_total // NUM_WORKERS
    n_chunks = n_per_w // CHUNK
    n_outer = n_chunks // NBUF
    mesh = plsc.VectorSubcoreMesh(core_axis_name="c", subcore_axis_name="s")

    @functools.partial(
        pl.kernel,
        mesh=mesh,
        out_type=jax.ShapeDtypeStruct((n_total, emb_dim), jnp.float32),
        scratch_types=[
            pltpu.VMEM((n_chunks, CHUNK), jnp.int32),
            pltpu.VMEM((NBUF, CHUNK, emb_dim), jnp.float32),
            pltpu.SemaphoreType.DMA((NBUF,)),
            pltpu.SemaphoreType.DMA((NBUF,)),
        ],
        compiler_params=pltpu.CompilerParams(use_tc_tiling_on_sc=False),
    )
    def body(idx_hbm, table_hbm, out_hbm, idx_v, rows_v, semg, semo):
        wid = lax.axis_index("s") * NUM_CORES + lax.axis_index("c")
        base = wid * n_per_w

        def gather(c, b):
            pltpu.async_copy(table_hbm.at[idx_v.at[c]], rows_v.at[b], semg.at[b])

        def gather_wait(c, b):
            pltpu.make_async_copy(
                table_hbm.at[idx_v.at[c]], rows_v.at[b], semg.at[b]
            ).wait()

        def put(c, b):
            pltpu.async_copy(
                rows_v.at[b], out_hbm.at[pl.ds(base + c * CHUNK, CHUNK)], semo.at[b]
            )

        def put_wait(c, b):
            pltpu.make_async_copy(
                rows_v.at[b], out_hbm.at[pl.ds(base + c * CHUNK, CHUNK)], semo.at[b]
            ).wait()

        pltpu.sync_copy(idx_hbm.at[pl.ds(wid * n_chunks, n_chunks)], idx_v)

        for b in range(NBUF):
            gather(b, b)

        def outer(o, carry):
            for b in range(NBUF):
                c = o * NBUF + b
                gather_wait(c, b)
                put(c, b)
            for b in range(NBUF):
                c = o * NBUF + b
                put_wait(c, b)
                gather(c + NBUF, b)
            return carry

        lax.fori_loop(0, n_outer - 1, outer, 0)

        for b in range(NBUF):
            c = (n_outer - 1) * NBUF + b
            gather_wait(c, b)
            put(c, b)
        for b in range(NBUF):
            c = (n_outer - 1) * NBUF + b
            put_wait(c, b)

    return body


def kernel(indices, table):
    batch, hist = indices.shape
    _, emb_dim = table.shape
    n_total = batch * hist
    idx2d = indices.reshape(n_total // CHUNK, CHUNK).astype(jnp.int32)
    out = _emb_lookup(n_total, emb_dim)(idx2d, table)
    return out.reshape(batch, hist, emb_dim)


# speedup vs baseline: 1.1935x; 1.1144x over previous
import functools

import jax
import jax.numpy as jnp
from jax import lax
from jax.experimental import pallas as pl
from jax.experimental.pallas import tpu as pltpu
from jax.experimental.pallas import tpu_sc as plsc

NUM_CORES = 2
NUM_SUBCORES = 16
NUM_WORKERS = NUM_CORES * NUM_SUBCORES
LANE = 128


def _emb_lookup(hist: int, n_hblk: int, emb_dim: int, n_bblk: int):
    n_dblk = emb_dim // 8
    mesh = plsc.VectorSubcoreMesh(core_axis_name="c", subcore_axis_name="s")

    @functools.partial(
        pl.kernel,
        mesh=mesh,
        out_type=jax.ShapeDtypeStruct(
            (hist, n_dblk, n_bblk, 8, LANE), jnp.float32
        ),
        scratch_types=[
            pltpu.VMEM((LANE,), jnp.int32),
            pltpu.VMEM((LANE,), jnp.int32),
            pltpu.VMEM((LANE, emb_dim), jnp.float32),
            pltpu.VMEM((LANE, emb_dim), jnp.float32),
            pltpu.VMEM((n_dblk, 8, LANE), jnp.float32),
            pltpu.VMEM((n_dblk, 8, LANE), jnp.float32),
            pltpu.SemaphoreType.DMA,
            pltpu.SemaphoreType.DMA,
            pltpu.SemaphoreType.DMA,
            pltpu.SemaphoreType.DMA,
        ],
        compiler_params=pltpu.CompilerParams(use_tc_tiling_on_sc=False, needs_layout_passes=False),
    )
    def body(idx_hbm, table_hbm, out_hbm, idx_a, idx_b, rows_a, rows_b,
             planes_a, planes_b, sga, sgb, soa, sob):
        w = lax.axis_index("s") * NUM_CORES + lax.axis_index("c")
        slots = ((idx_a, rows_a, planes_a, sga, soa),
                 (idx_b, rows_b, planes_b, sgb, sob))

        def stage(h, slot):
            idx_v, rows_v, _, sg, _ = slot
            pltpu.sync_copy(idx_hbm.at[h // 8, w, h % 8], idx_v)
            pltpu.async_copy(table_hbm.at[idx_v], rows_v, sg)

        def gather_wait(slot):
            idx_v, rows_v, _, sg, _ = slot
            pltpu.make_async_copy(table_hbm.at[idx_v], rows_v, sg).wait()

        def put(h, slot):
            _, _, planes_v, _, so = slot
            pltpu.async_copy(planes_v, out_hbm.at[h, :, w], so)

        def put_wait(h, slot):
            _, _, planes_v, _, so = slot
            pltpu.make_async_copy(planes_v, out_hbm.at[h, :, w], so).wait()

        def transpose(slot):
            _, rows_v, planes_v, _, _ = slot
            base = lax.iota(jnp.int32, 16)
            for d in range(emb_dim):
                col = jnp.full((16,), d, jnp.int32)
                for j0 in range(LANE // 16):
                    ridx = base + (j0 * 16)
                    v = plsc.load_gather(rows_v, [ridx, col])
                    planes_v[d // 8, d % 8, pl.ds(j0 * 16, 16)] = v

        stage(0, slots[0])

        def outer(o, carry):
            for b in (0, 1):
                h = o * 2 + b
                nxt = h + 1

                @pl.when(nxt < hist)
                def _():
                    stage(nxt, slots[1 - b])

                gather_wait(slots[b])

                @pl.when(h >= 2)
                def _():
                    put_wait(h - 2, slots[b])

                transpose(slots[b])
                put(h, slots[b])
            return carry

        lax.fori_loop(0, hist // 2, outer, 0)
        put_wait(hist - 2, slots[0])
        put_wait(hist - 1, slots[1])

    return body


def kernel(indices, table):
    batch, hist = indices.shape
    _, emb_dim = table.shape
    n_bblk = batch // LANE
    hist_pad = -(-hist // 8) * 8
    n_hblk = hist_pad // 8
    idx_p = jnp.pad(indices.astype(jnp.int32), ((0, 0), (0, hist_pad - hist)))
    idx4 = (
        idx_p.T.reshape(n_hblk, 8, n_bblk, LANE).transpose(0, 2, 1, 3)
    )
    out5 = _emb_lookup(hist, n_hblk, emb_dim, n_bblk)(idx4, table)
    out = out5.transpose(2, 4, 0, 1, 3).reshape(batch, hist, emb_dim)
    return out
